# baseline (speedup 1.0000x reference)
import jax
import jax.numpy as jnp
from jax.experimental import pallas as pl
from jax.experimental.pallas import tpu as pltpu

_T = 0.1
_K = 5
_N = 4096
_D = 128
_BLK = 256
_LANES = 128
_L2E = 1.4426950408889634
_LN2 = 0.6931471805599453
_SCALE = (_L2E / _T) ** 0.5


def _normalize_kernel(z_ref, zn_ref):
    z = z_ref[...]
    ss = jnp.sum(z * z, axis=1, keepdims=True)
    zn_ref[...] = z * (jax.lax.rsqrt(jnp.maximum(ss, 1e-24)) * _SCALE)


def _merge22(x0, x1, y0, y1):
    z0 = jnp.maximum(x0, y0)
    l0 = jnp.minimum(x0, y0)
    w1 = jnp.maximum(x1, y1)
    z3 = jnp.minimum(x1, y1)
    z1 = jnp.maximum(l0, w1)
    z2 = jnp.minimum(l0, w1)
    return z0, z1, z2, z3


def _merge44_top6(a, b):
    e = _merge22(a[0], a[2], b[0], b[2])
    o = _merge22(a[1], a[3], b[1], b[3])
    return [
        e[0],
        jnp.maximum(e[1], o[0]),
        jnp.minimum(e[1], o[0]),
        jnp.maximum(e[2], o[1]),
        jnp.minimum(e[2], o[1]),
        jnp.maximum(e[3], o[2]),
    ]


def _merge66_top6(a, b):
    out = [jnp.maximum(a[0], b[0])]
    for i in range(1, 6):
        r = jnp.maximum(a[i], b[i])
        for j in range(i):
            r = jnp.maximum(r, jnp.minimum(a[j], b[i - 1 - j]))
        out.append(r)
    return out


def _top6_of_s2(s2):
    s4 = [
        _merge22(s2[2 * i][0], s2[2 * i][1], s2[2 * i + 1][0], s2[2 * i + 1][1])
        for i in range(8)
    ]
    s6 = [_merge44_top6(s4[2 * i], s4[2 * i + 1]) for i in range(4)]
    return _merge66_top6(
        _merge66_top6(s6[0], s6[1]), _merge66_top6(s6[2], s6[3])
    )


def _and_loss_kernel(zn_ref, out_ref):
    i = pl.program_id(0)
    zn = zn_ref[...]
    zb = zn_ref[pl.ds(i * _BLK, _BLK), :]
    logits = jnp.dot(zb, zn.T, preferred_element_type=jnp.float32)

    neg = jnp.float32(-jnp.inf)
    e_acc = jnp.zeros((_BLK, _LANES), jnp.float32)
    s2 = []
    for c in range(_N // (2 * _LANES)):
        x0 = logits[:, (2 * c) * _LANES:(2 * c + 1) * _LANES]
        x1 = logits[:, (2 * c + 1) * _LANES:(2 * c + 2) * _LANES]
        e_acc = e_acc + (jnp.exp2(x0) + jnp.exp2(x1))
        s2.append((jnp.maximum(x0, x1), jnp.minimum(x0, x1)))
    t = _top6_of_s2(s2)

    m0 = jnp.max(t[0], axis=1, keepdims=True)
    s_full = jnp.sum(e_acc, axis=1, keepdims=True)
    lse = jnp.log2(s_full - jnp.exp2(m0))

    mask0 = t[0] == m0
    c0 = jnp.sum(mask0.astype(jnp.float32), axis=1, keepdims=True)
    u = [jnp.where(mask0, t[j + 1], t[j]) for j in range(_K)]

    rem = jnp.minimum(c0 - 1.0, _K * 1.0)
    acc = rem
    for k in range(_K):
        vk = jnp.max(u[0], axis=1, keepdims=True)
        mk = u[0] == vk
        ck = jnp.sum(mk.astype(jnp.float32), axis=1, keepdims=True)
        w = jnp.minimum(ck, _K - rem)
        rem = rem + w
        acc = acc + w * jnp.exp2(vk - m0)
        if k < _K - 1:
            u = [jnp.where(mk, u[j + 1], u[j]) for j in range(_K - 1)] + [
                jnp.where(mk, neg, u[_K - 1])
            ]
    num = m0 + jnp.log2(acc)

    part = jnp.sum(num - lse, keepdims=True)
    out_ref[...] = jnp.broadcast_to(part[None], (1, 1, _LANES))


def kernel(z):
    zn = pl.pallas_call(
        _normalize_kernel,
        out_shape=jax.ShapeDtypeStruct((_N, _D), jnp.float32),
    )(z)
    partials = pl.pallas_call(
        _and_loss_kernel,
        grid=(_N // _BLK,),
        in_specs=[pl.BlockSpec((_N, _D), lambda i: (0, 0))],
        out_specs=pl.BlockSpec((1, 1, _LANES), lambda i: (i, 0, 0)),
        out_shape=jax.ShapeDtypeStruct((_N // _BLK, 1, _LANES), jnp.float32),
        compiler_params=pltpu.CompilerParams(
            dimension_semantics=("parallel",),
        ),
    )(zn)
    return -jnp.sum(partials[:, 0, 0]) * (_LN2 / _N)


# speedup vs baseline: 1.0881x; 1.0881x over previous
import jax
import jax.numpy as jnp
from jax.experimental import pallas as pl
from jax.experimental.pallas import tpu as pltpu

_T = 0.1
_K = 5
_N = 4096
_D = 128
_BLK = 512
_LANES = 128
_L2E = 1.4426950408889634
_LN2 = 0.6931471805599453
_SCALE = (_L2E / _T) ** 0.5


def _normalize_kernel(z_ref, zn_ref):
    z = z_ref[...]
    ss = jnp.sum(z * z, axis=1, keepdims=True)
    zn_ref[...] = z * (jax.lax.rsqrt(jnp.maximum(ss, 1e-24)) * _SCALE)


def _merge22(x0, x1, y0, y1):
    z0 = jnp.maximum(x0, y0)
    l0 = jnp.minimum(x0, y0)
    w1 = jnp.maximum(x1, y1)
    z3 = jnp.minimum(x1, y1)
    z1 = jnp.maximum(l0, w1)
    z2 = jnp.minimum(l0, w1)
    return z0, z1, z2, z3


def _merge44_top6(a, b):
    e = _merge22(a[0], a[2], b[0], b[2])
    o = _merge22(a[1], a[3], b[1], b[3])
    return [
        e[0],
        jnp.maximum(e[1], o[0]),
        jnp.minimum(e[1], o[0]),
        jnp.maximum(e[2], o[1]),
        jnp.minimum(e[2], o[1]),
        jnp.maximum(e[3], o[2]),
    ]


def _merge66_top6(a, b):
    out = [jnp.maximum(a[0], b[0])]
    for i in range(1, 6):
        r = jnp.maximum(a[i], b[i])
        for j in range(i):
            r = jnp.maximum(r, jnp.minimum(a[j], b[i - 1 - j]))
        out.append(r)
    return out


def _top6_of_s2(s2):
    s4 = [
        _merge22(s2[2 * i][0], s2[2 * i][1], s2[2 * i + 1][0], s2[2 * i + 1][1])
        for i in range(8)
    ]
    s6 = [_merge44_top6(s4[2 * i], s4[2 * i + 1]) for i in range(4)]
    return _merge66_top6(
        _merge66_top6(s6[0], s6[1]), _merge66_top6(s6[2], s6[3])
    )


def _and_loss_kernel(zn_ref, out_ref):
    i = pl.program_id(0)
    zn = zn_ref[...]
    zb = zn_ref[pl.ds(i * _BLK, _BLK), :]
    logits = jnp.dot(zb, zn.T, preferred_element_type=jnp.float32)

    neg = jnp.float32(-jnp.inf)
    e_acc = jnp.zeros((_BLK, _LANES), jnp.float32)
    xs = []
    for c in range(_N // _LANES):
        x = logits[:, c * _LANES:(c + 1) * _LANES]
        e_acc = e_acc + jnp.exp2(x)
        xs.append(x)
    s2 = [
        (jnp.maximum(xs[2 * i], xs[2 * i + 1]),
         jnp.minimum(xs[2 * i], xs[2 * i + 1]))
        for i in range(_N // (2 * _LANES))
    ]
    t = _top6_of_s2(s2)

    m0 = jnp.max(t[0], axis=1, keepdims=True)
    s_full = jnp.sum(e_acc, axis=1, keepdims=True)
    lse = jnp.log2(s_full - jnp.exp2(m0))

    mask0 = t[0] == m0
    c0 = jnp.sum(mask0.astype(jnp.float32), axis=1, keepdims=True)
    u = [jnp.where(mask0, t[j + 1], t[j]) for j in range(_K)]

    rem = jnp.minimum(c0 - 1.0, _K * 1.0)
    acc = rem
    for k in range(_K):
        vk = jnp.max(u[0], axis=1, keepdims=True)
        mk = u[0] == vk
        ck = jnp.sum(mk.astype(jnp.float32), axis=1, keepdims=True)
        w = jnp.minimum(ck, _K - rem)
        rem = rem + w
        acc = acc + w * jnp.exp2(vk - m0)
        if k < _K - 1:
            u = [jnp.where(mk, u[j + 1], u[j]) for j in range(_K - 1)] + [
                jnp.where(mk, neg, u[_K - 1])
            ]
    num = m0 + jnp.log2(acc)

    part = jnp.sum(num - lse, keepdims=True)
    out_ref[...] = jnp.broadcast_to(part[None], (1, 1, _LANES))


def kernel(z):
    zn = pl.pallas_call(
        _normalize_kernel,
        out_shape=jax.ShapeDtypeStruct((_N, _D), jnp.float32),
    )(z)
    partials = pl.pallas_call(
        _and_loss_kernel,
        grid=(_N // _BLK,),
        in_specs=[pl.BlockSpec((_N, _D), lambda i: (0, 0))],
        out_specs=pl.BlockSpec((1, 1, _LANES), lambda i: (i, 0, 0)),
        out_shape=jax.ShapeDtypeStruct((_N // _BLK, 1, _LANES), jnp.float32),
        compiler_params=pltpu.CompilerParams(
            dimension_semantics=("parallel",),
        ),
    )(zn)
    return -jnp.sum(partials[:, 0, 0]) * (_LN2 / _N)


# speedup vs baseline: 1.0907x; 1.0023x over previous
import jax
import jax.numpy as jnp
from jax.experimental import pallas as pl
from jax.experimental.pallas import tpu as pltpu

_T = 0.1
_K = 5
_N = 4096
_D = 128
_BLK = 1024
_LANES = 128
_L2E = 1.4426950408889634
_LN2 = 0.6931471805599453
_SCALE = (_L2E / _T) ** 0.5


def _normalize_kernel(z_ref, zn_ref):
    z = z_ref[...]
    ss = jnp.sum(z * z, axis=1, keepdims=True)
    zn_ref[...] = z * (jax.lax.rsqrt(jnp.maximum(ss, 1e-24)) * _SCALE)


def _merge22(x0, x1, y0, y1):
    z0 = jnp.maximum(x0, y0)
    l0 = jnp.minimum(x0, y0)
    w1 = jnp.maximum(x1, y1)
    z3 = jnp.minimum(x1, y1)
    z1 = jnp.maximum(l0, w1)
    z2 = jnp.minimum(l0, w1)
    return z0, z1, z2, z3


def _merge44_top6(a, b):
    e = _merge22(a[0], a[2], b[0], b[2])
    o = _merge22(a[1], a[3], b[1], b[3])
    return [
        e[0],
        jnp.maximum(e[1], o[0]),
        jnp.minimum(e[1], o[0]),
        jnp.maximum(e[2], o[1]),
        jnp.minimum(e[2], o[1]),
        jnp.maximum(e[3], o[2]),
    ]


def _merge66_top6(a, b):
    out = [jnp.maximum(a[0], b[0])]
    for i in range(1, 6):
        r = jnp.maximum(a[i], b[i])
        for j in range(i):
            r = jnp.maximum(r, jnp.minimum(a[j], b[i - 1 - j]))
        out.append(r)
    return out


def _top6_of_s2(s2):
    s4 = [
        _merge22(s2[2 * i][0], s2[2 * i][1], s2[2 * i + 1][0], s2[2 * i + 1][1])
        for i in range(8)
    ]
    s6 = [_merge44_top6(s4[2 * i], s4[2 * i + 1]) for i in range(4)]
    return _merge66_top6(
        _merge66_top6(s6[0], s6[1]), _merge66_top6(s6[2], s6[3])
    )


def _and_loss_kernel(zn_ref, out_ref):
    i = pl.program_id(0)
    zn = zn_ref[...]
    zb = zn_ref[pl.ds(i * _BLK, _BLK), :]
    logits = jnp.dot(zb, zn.T, preferred_element_type=jnp.float32)

    neg = jnp.float32(-jnp.inf)
    e_acc = jnp.zeros((_BLK, _LANES), jnp.float32)
    xs = []
    for c in range(_N // _LANES):
        x = logits[:, c * _LANES:(c + 1) * _LANES]
        e_acc = e_acc + jnp.exp2(x)
        xs.append(x)
    s2 = [
        (jnp.maximum(xs[2 * i], xs[2 * i + 1]),
         jnp.minimum(xs[2 * i], xs[2 * i + 1]))
        for i in range(_N // (2 * _LANES))
    ]
    t = _top6_of_s2(s2)

    m0 = jnp.max(t[0], axis=1, keepdims=True)
    s_full = jnp.sum(e_acc, axis=1, keepdims=True)
    lse = jnp.log2(s_full - jnp.exp2(m0))

    mask0 = t[0] == m0
    c0 = jnp.sum(mask0.astype(jnp.float32), axis=1, keepdims=True)
    u = [jnp.where(mask0, t[j + 1], t[j]) for j in range(_K)]

    rem = jnp.minimum(c0 - 1.0, _K * 1.0)
    acc = rem
    for k in range(_K):
        vk = jnp.max(u[0], axis=1, keepdims=True)
        mk = u[0] == vk
        ck = jnp.sum(mk.astype(jnp.float32), axis=1, keepdims=True)
        w = jnp.minimum(ck, _K - rem)
        rem = rem + w
        acc = acc + w * jnp.exp2(vk - m0)
        if k < _K - 1:
            u = [jnp.where(mk, u[j + 1], u[j]) for j in range(_K - 1)] + [
                jnp.where(mk, neg, u[_K - 1])
            ]
    num = m0 + jnp.log2(acc)

    part = jnp.sum(num - lse, keepdims=True)
    out_ref[...] = jnp.broadcast_to(part[None], (1, 1, _LANES))


def kernel(z):
    zn = pl.pallas_call(
        _normalize_kernel,
        out_shape=jax.ShapeDtypeStruct((_N, _D), jnp.float32),
    )(z)
    partials = pl.pallas_call(
        _and_loss_kernel,
        grid=(_N // _BLK,),
        in_specs=[pl.BlockSpec((_N, _D), lambda i: (0, 0))],
        out_specs=pl.BlockSpec((1, 1, _LANES), lambda i: (i, 0, 0)),
        out_shape=jax.ShapeDtypeStruct((_N // _BLK, 1, _LANES), jnp.float32),
        compiler_params=pltpu.CompilerParams(
            dimension_semantics=("parallel",),
        ),
    )(zn)
    return -jnp.sum(partials[:, 0, 0]) * (_LN2 / _N)


# speedup vs baseline: 1.1311x; 1.0371x over previous
import jax
import jax.numpy as jnp
from jax.experimental import pallas as pl
from jax.experimental.pallas import tpu as pltpu

_T = 0.1
_K = 5
_N = 4096
_D = 128
_BLK = 2048
_LANES = 128
_L2E = 1.4426950408889634
_LN2 = 0.6931471805599453
_SCALE = (_L2E / _T) ** 0.5


def _normalize_kernel(z_ref, zn_ref):
    z = z_ref[...]
    ss = jnp.sum(z * z, axis=1, keepdims=True)
    zn_ref[...] = z * (jax.lax.rsqrt(jnp.maximum(ss, 1e-24)) * _SCALE)


def _merge22(x0, x1, y0, y1):
    z0 = jnp.maximum(x0, y0)
    l0 = jnp.minimum(x0, y0)
    w1 = jnp.maximum(x1, y1)
    z3 = jnp.minimum(x1, y1)
    z1 = jnp.maximum(l0, w1)
    z2 = jnp.minimum(l0, w1)
    return z0, z1, z2, z3


def _merge44_top6(a, b):
    e = _merge22(a[0], a[2], b[0], b[2])
    o = _merge22(a[1], a[3], b[1], b[3])
    return [
        e[0],
        jnp.maximum(e[1], o[0]),
        jnp.minimum(e[1], o[0]),
        jnp.maximum(e[2], o[1]),
        jnp.minimum(e[2], o[1]),
        jnp.maximum(e[3], o[2]),
    ]


def _merge66_top6(a, b):
    out = [jnp.maximum(a[0], b[0])]
    for i in range(1, 6):
        r = jnp.maximum(a[i], b[i])
        for j in range(i):
            r = jnp.maximum(r, jnp.minimum(a[j], b[i - 1 - j]))
        out.append(r)
    return out


def _top6_of_s2(s2):
    s4 = [
        _merge22(s2[2 * i][0], s2[2 * i][1], s2[2 * i + 1][0], s2[2 * i + 1][1])
        for i in range(8)
    ]
    s6 = [_merge44_top6(s4[2 * i], s4[2 * i + 1]) for i in range(4)]
    return _merge66_top6(
        _merge66_top6(s6[0], s6[1]), _merge66_top6(s6[2], s6[3])
    )


def _and_loss_kernel(zn_ref, out_ref):
    i = pl.program_id(0)
    zn = zn_ref[...]
    zb = zn_ref[pl.ds(i * _BLK, _BLK), :]
    logits = jnp.dot(zb, zn.T, preferred_element_type=jnp.float32)

    neg = jnp.float32(-jnp.inf)
    e_acc = jnp.zeros((_BLK, _LANES), jnp.float32)
    xs = []
    for c in range(_N // _LANES):
        x = logits[:, c * _LANES:(c + 1) * _LANES]
        e_acc = e_acc + jnp.exp2(x)
        xs.append(x)
    s2 = [
        (jnp.maximum(xs[2 * i], xs[2 * i + 1]),
         jnp.minimum(xs[2 * i], xs[2 * i + 1]))
        for i in range(_N // (2 * _LANES))
    ]
    t = _top6_of_s2(s2)

    m0 = jnp.max(t[0], axis=1, keepdims=True)
    s_full = jnp.sum(e_acc, axis=1, keepdims=True)
    lse = jnp.log2(s_full - jnp.exp2(m0))

    mask0 = t[0] == m0
    c0 = jnp.sum(mask0.astype(jnp.float32), axis=1, keepdims=True)
    u = [jnp.where(mask0, t[j + 1], t[j]) for j in range(_K)]

    rem = jnp.minimum(c0 - 1.0, _K * 1.0)
    acc = rem
    for k in range(_K):
        vk = jnp.max(u[0], axis=1, keepdims=True)
        mk = u[0] == vk
        ck = jnp.sum(mk.astype(jnp.float32), axis=1, keepdims=True)
        w = jnp.minimum(ck, _K - rem)
        rem = rem + w
        acc = acc + w * jnp.exp2(vk - m0)
        if k < _K - 1:
            u = [jnp.where(mk, u[j + 1], u[j]) for j in range(_K - 1)] + [
                jnp.where(mk, neg, u[_K - 1])
            ]
    num = m0 + jnp.log2(acc)

    part = jnp.sum(num - lse, keepdims=True)
    out_ref[...] = jnp.broadcast_to(part[None], (1, 1, _LANES))


def kernel(z):
    zn = pl.pallas_call(
        _normalize_kernel,
        out_shape=jax.ShapeDtypeStruct((_N, _D), jnp.float32),
    )(z)
    partials = pl.pallas_call(
        _and_loss_kernel,
        grid=(_N // _BLK,),
        in_specs=[pl.BlockSpec((_N, _D), lambda i: (0, 0))],
        out_specs=pl.BlockSpec((1, 1, _LANES), lambda i: (i, 0, 0)),
        out_shape=jax.ShapeDtypeStruct((_N // _BLK, 1, _LANES), jnp.float32),
        compiler_params=pltpu.CompilerParams(
            dimension_semantics=("parallel",),
        ),
    )(zn)
    return -jnp.sum(partials[:, 0, 0]) * (_LN2 / _N)
